# Initial kernel scaffold; baseline (speedup 1.0000x reference)
import jax
import jax.numpy as jnp
from jax.experimental import pallas as pl


def kernel(x, router_w, W1, W3, W2, sw1, sw3, sw2, experts_bias):
    raise NotImplementedError("write your pallas kernel here")


# speedup vs baseline: 1.0942x; 1.0942x over previous
import functools

import jax
import jax.numpy as jnp
from jax import lax
from jax.experimental import pallas as pl
from jax.experimental.pallas import tpu as pltpu
from jax.experimental.pallas import tpu_sc as plsc

DIM = 2048
HID = 1024
NE = 8
SEQ = 2048
BLK = 128
NBLK = 2 * SEQ // BLK + 8
NPAD = NBLK * BLK
NC = 2
NS = 16
NW = NC * NS
TPW = SEQ // NW
SUB = 16
SB = 512


def _route_body(x_ref, rw_ref, b_ref, pos0_ref, pos1_ref, w0_ref, w1_ref,
                be_ref, na_ref):
    x = x_ref[...]
    logits = lax.dot_general(x, rw_ref[...], (((1,), (1,)), ((), ())),
                             preferred_element_type=jnp.float32)
    logits = logits + b_ref[...]
    eidx = lax.broadcasted_iota(jnp.int32, (SEQ, NE), 1)
    m0 = jnp.max(logits, axis=1, keepdims=True)
    i0 = jnp.min(jnp.where(logits == m0, eidx, NE), axis=1, keepdims=True)
    sel0 = eidx == i0
    l2 = jnp.where(sel0, -jnp.inf, logits)
    m1 = jnp.max(l2, axis=1, keepdims=True)
    i1 = jnp.min(jnp.where(l2 == m1, eidx, NE), axis=1, keepdims=True)
    sel1 = eidx == i1
    d = jnp.exp(m1 - m0)
    inv = 1.0 / (1.0 + d)
    w0_ref[...] = inv
    w1_ref[...] = d * inv
    counts = sel0.astype(jnp.float32) + sel1.astype(jnp.float32)
    r_i = lax.broadcasted_iota(jnp.int32, (SEQ, SEQ), 0)
    c_i = lax.broadcasted_iota(jnp.int32, (SEQ, SEQ), 1)
    tri = (c_i < r_i).astype(jnp.float32)
    csum = lax.dot_general(tri, counts, (((1,), (0,)), ((), ())),
                           preferred_element_type=jnp.float32)
    tot = jnp.sum(counts, axis=0, keepdims=True)
    padded = jnp.ceil(tot / BLK) * BLK
    u_r = lax.broadcasted_iota(jnp.int32, (NE, NE), 0)
    u_c = lax.broadcasted_iota(jnp.int32, (NE, NE), 1)
    upper = (u_r < u_c).astype(jnp.float32)
    gbase = lax.dot_general(padded, upper, (((1,), (0,)), ((), ())),
                            preferred_element_type=jnp.float32)
    slot = gbase + csum
    pos0_ref[...] = jnp.sum(jnp.where(sel0, slot, 0.0), axis=1,
                            keepdims=True).astype(jnp.int32)
    pos1_ref[...] = jnp.sum(jnp.where(sel1, slot, 0.0), axis=1,
                            keepdims=True).astype(jnp.int32)
    cum_in = gbase + padded
    total = jnp.sum(padded)
    bpos = (lax.broadcasted_iota(jnp.int32, (NBLK, NE), 0) * BLK
            ).astype(jnp.float32)
    be_raw = jnp.sum((bpos >= cum_in).astype(jnp.int32), axis=1,
                     keepdims=True)
    last_e = jnp.sum(((total - BLK) >= cum_in).astype(jnp.int32))
    be_ref[...] = jnp.minimum(be_raw, last_e)
    na_ref[...] = jnp.full((1, 1), 0, jnp.int32) + (
        total.astype(jnp.int32) // BLK)


def _route(xf, router_w, bias2d):
    return pl.pallas_call(
        _route_body,
        out_shape=[
            jax.ShapeDtypeStruct((SEQ, 1), jnp.int32),
            jax.ShapeDtypeStruct((SEQ, 1), jnp.int32),
            jax.ShapeDtypeStruct((SEQ, 1), jnp.float32),
            jax.ShapeDtypeStruct((SEQ, 1), jnp.float32),
            jax.ShapeDtypeStruct((NBLK, 1), jnp.int32),
            jax.ShapeDtypeStruct((1, 1), jnp.int32),
        ],
    )(xf, router_w, bias2d)


def _dispatch_body(x_hbm, p0_hbm, p1_hbm, xs_hbm, rows_v, i0_v, i1_v, sem):
    wid = lax.axis_index("s") * NC + lax.axis_index("c")
    base = wid * TPW

    def chunk(j, carry):
        b = base + j * SUB
        pltpu.sync_copy(x_hbm.at[pl.ds(b, SUB)], rows_v)
        pltpu.sync_copy(p0_hbm.at[pl.ds(b, SUB)], i0_v)
        pltpu.sync_copy(p1_hbm.at[pl.ds(b, SUB)], i1_v)
        c0 = pltpu.async_copy(rows_v, xs_hbm.at[i0_v], sem)
        c1 = pltpu.async_copy(rows_v, xs_hbm.at[i1_v], sem)
        c0.wait()
        c1.wait()
        return carry

    lax.fori_loop(0, TPW // SUB, chunk, 0)


@functools.cache
def _dispatch():
    return pl.kernel(
        _dispatch_body,
        out_type=jax.ShapeDtypeStruct((NPAD, DIM), jnp.float32),
        mesh=plsc.VectorSubcoreMesh(core_axis_name="c", subcore_axis_name="s",
                                    num_cores=NC, num_subcores=NS),
        scratch_types=[
            pltpu.VMEM((SUB, DIM), jnp.float32),
            pltpu.VMEM((SUB,), jnp.int32),
            pltpu.VMEM((SUB,), jnp.int32),
            pltpu.SemaphoreType.DMA,
        ],
    )


def _ff_block(xb, w1, w3, w2):
    a = lax.dot_general(xb, w1, (((1,), (1,)), ((), ())),
                        preferred_element_type=jnp.float32)
    g = lax.dot_general(xb, w3, (((1,), (1,)), ((), ())),
                        preferred_element_type=jnp.float32)
    h = (a * jax.nn.sigmoid(a)) * g
    return lax.dot_general(h.astype(jnp.bfloat16), w2,
                           (((1,), (1,)), ((), ())),
                           preferred_element_type=jnp.float32)


def _gemm_body(bk_ref, xs_ref, w1_ref, w3_ref, w2_ref, o_ref):
    i = pl.program_id(0)

    @pl.when(i < bk_ref[NBLK])
    def _():
        xb = xs_ref[...].astype(jnp.bfloat16)
        o_ref[...] = _ff_block(xb, w1_ref[0], w3_ref[0], w2_ref[0])


def _gemm(bk, xs, w1b, w3b, w2b):
    grid_spec = pltpu.PrefetchScalarGridSpec(
        num_scalar_prefetch=1,
        grid=(NBLK,),
        in_specs=[
            pl.BlockSpec((BLK, DIM), lambda i, bk: (i, 0)),
            pl.BlockSpec((1, HID, DIM), lambda i, bk: (bk[i], 0, 0)),
            pl.BlockSpec((1, HID, DIM), lambda i, bk: (bk[i], 0, 0)),
            pl.BlockSpec((1, DIM, HID), lambda i, bk: (bk[i], 0, 0)),
        ],
        out_specs=pl.BlockSpec((BLK, DIM), lambda i, bk: (i, 0)),
    )
    return pl.pallas_call(
        _gemm_body,
        grid_spec=grid_spec,
        out_shape=jax.ShapeDtypeStruct((NPAD, DIM), jnp.float32),
    )(bk, xs, w1b, w3b, w2b)


def _shared_body(x_ref, w1_ref, w3_ref, w2_ref, o_ref):
    xb = x_ref[...].astype(jnp.bfloat16)
    o_ref[...] = _ff_block(xb, w1_ref[...], w3_ref[...], w2_ref[...])


def _shared(xf, sw1b, sw3b, sw2b):
    return pl.pallas_call(
        _shared_body,
        grid=(SEQ // SB,),
        in_specs=[
            pl.BlockSpec((SB, DIM), lambda i: (i, 0)),
            pl.BlockSpec((HID, DIM), lambda i: (0, 0)),
            pl.BlockSpec((HID, DIM), lambda i: (0, 0)),
            pl.BlockSpec((DIM, HID), lambda i: (0, 0)),
        ],
        out_specs=pl.BlockSpec((SB, DIM), lambda i: (i, 0)),
        out_shape=jax.ShapeDtypeStruct((SEQ, DIM), jnp.float32),
    )(xf, sw1b, sw3b, sw2b)


def _combine_body(ys_hbm, sh_hbm, p0_hbm, p1_hbm, w0_hbm, w1_hbm, out_hbm,
                  acc_v, r0_v, r1_v, w0_v, w1_v, i0_v, i1_v, sem):
    wid = lax.axis_index("s") * NC + lax.axis_index("c")
    base = wid * TPW

    def chunk(j, carry):
        b = base + j * SUB
        pltpu.sync_copy(sh_hbm.at[pl.ds(b, SUB)], acc_v)
        pltpu.sync_copy(p0_hbm.at[pl.ds(b, SUB)], i0_v)
        pltpu.sync_copy(p1_hbm.at[pl.ds(b, SUB)], i1_v)
        pltpu.sync_copy(w0_hbm.at[pl.ds(b, SUB)], w0_v)
        pltpu.sync_copy(w1_hbm.at[pl.ds(b, SUB)], w1_v)
        c0 = pltpu.async_copy(ys_hbm.at[i0_v], r0_v, sem)
        c1 = pltpu.async_copy(ys_hbm.at[i1_v], r1_v, sem)
        c0.wait()
        c1.wait()
        w0vec = w0_v[...]
        w1vec = w1_v[...]
        ws = [(w0vec[t], w1vec[t]) for t in range(SUB)]

        def col(c, carry2):
            s = pl.ds(c * 16, 16)
            for t in range(SUB):
                a, g = ws[t]
                acc_v[t, s] = acc_v[t, s] + a * r0_v[t, s] + g * r1_v[t, s]
            return carry2

        lax.fori_loop(0, DIM // 16, col, 0)
        pltpu.sync_copy(acc_v, out_hbm.at[pl.ds(b, SUB)])
        return carry

    lax.fori_loop(0, TPW // SUB, chunk, 0)


@functools.cache
def _combine():
    return pl.kernel(
        _combine_body,
        out_type=jax.ShapeDtypeStruct((SEQ, DIM), jnp.float32),
        mesh=plsc.VectorSubcoreMesh(core_axis_name="c", subcore_axis_name="s",
                                    num_cores=NC, num_subcores=NS),
        scratch_types=[
            pltpu.VMEM((SUB, DIM), jnp.float32),
            pltpu.VMEM((SUB, DIM), jnp.float32),
            pltpu.VMEM((SUB, DIM), jnp.float32),
            pltpu.VMEM((SUB,), jnp.float32),
            pltpu.VMEM((SUB,), jnp.float32),
            pltpu.VMEM((SUB,), jnp.int32),
            pltpu.VMEM((SUB,), jnp.int32),
            pltpu.SemaphoreType.DMA,
        ],
    )


@jax.jit
def kernel(x, router_w, W1, W3, W2, sw1, sw3, sw2, experts_bias):
    xf = x.reshape(SEQ, DIM)
    pos0, pos1, w0, w1, be, na = _route(xf, router_w,
                                        experts_bias.reshape(1, NE))
    bk = jnp.concatenate([be.reshape(NBLK), na.reshape(1)])
    p0 = pos0.reshape(SEQ)
    p1 = pos1.reshape(SEQ)
    xs = _dispatch()(xf, p0, p1)
    ys = _gemm(bk, xs, W1.astype(jnp.bfloat16), W3.astype(jnp.bfloat16),
               W2.astype(jnp.bfloat16))
    sh = _shared(xf, sw1.astype(jnp.bfloat16), sw3.astype(jnp.bfloat16),
                 sw2.astype(jnp.bfloat16))
    out = _combine()(ys, sh, p0, p1, w0.reshape(SEQ), w1.reshape(SEQ))
    return out.reshape(x.shape)


# speedup vs baseline: 1.3477x; 1.2317x over previous
import functools

import jax
import jax.numpy as jnp
from jax import lax
from jax.experimental import pallas as pl
from jax.experimental.pallas import tpu as pltpu
from jax.experimental.pallas import tpu_sc as plsc

DIM = 2048
HID = 1024
NE = 8
SEQ = 2048
BLK = 128
NBLK = 2 * SEQ // BLK + 8
NPAD = NBLK * BLK
NC = 2
NS = 16
NW = NC * NS
TPW = SEQ // NW
SUB = 16
SB = 512


def _route_body(x_ref, rw_ref, b_ref, pos0_ref, pos1_ref, w0_ref, w1_ref,
                be_ref, na_ref):
    x = x_ref[...]
    logits = lax.dot_general(x, rw_ref[...], (((1,), (1,)), ((), ())),
                             preferred_element_type=jnp.float32)
    logits = logits + b_ref[...]
    eidx = lax.broadcasted_iota(jnp.int32, (SEQ, NE), 1)
    m0 = jnp.max(logits, axis=1, keepdims=True)
    i0 = jnp.min(jnp.where(logits == m0, eidx, NE), axis=1, keepdims=True)
    sel0 = eidx == i0
    l2 = jnp.where(sel0, -jnp.inf, logits)
    m1 = jnp.max(l2, axis=1, keepdims=True)
    i1 = jnp.min(jnp.where(l2 == m1, eidx, NE), axis=1, keepdims=True)
    sel1 = eidx == i1
    d = jnp.exp(m1 - m0)
    inv = 1.0 / (1.0 + d)
    w0_ref[...] = inv
    w1_ref[...] = d * inv
    counts = sel0.astype(jnp.float32) + sel1.astype(jnp.float32)
    r_i = lax.broadcasted_iota(jnp.int32, (SEQ, SEQ), 0)
    c_i = lax.broadcasted_iota(jnp.int32, (SEQ, SEQ), 1)
    tri = (c_i < r_i).astype(jnp.float32)
    csum = lax.dot_general(tri, counts, (((1,), (0,)), ((), ())),
                           preferred_element_type=jnp.float32)
    tot = jnp.sum(counts, axis=0, keepdims=True)
    padded = jnp.ceil(tot / BLK) * BLK
    u_r = lax.broadcasted_iota(jnp.int32, (NE, NE), 0)
    u_c = lax.broadcasted_iota(jnp.int32, (NE, NE), 1)
    upper = (u_r < u_c).astype(jnp.float32)
    gbase = lax.dot_general(padded, upper, (((1,), (0,)), ((), ())),
                            preferred_element_type=jnp.float32)
    slot = gbase + csum
    pos0_ref[...] = jnp.sum(jnp.where(sel0, slot, 0.0), axis=1,
                            keepdims=True).astype(jnp.int32)
    pos1_ref[...] = jnp.sum(jnp.where(sel1, slot, 0.0), axis=1,
                            keepdims=True).astype(jnp.int32)
    cum_in = gbase + padded
    total = jnp.sum(padded)
    bpos = (lax.broadcasted_iota(jnp.int32, (NBLK, NE), 0) * BLK
            ).astype(jnp.float32)
    be_raw = jnp.sum((bpos >= cum_in).astype(jnp.int32), axis=1,
                     keepdims=True)
    last_e = jnp.sum(((total - BLK) >= cum_in).astype(jnp.int32))
    be_ref[...] = jnp.minimum(be_raw, last_e)
    na_ref[...] = jnp.full((1, 1), 0, jnp.int32) + (
        total.astype(jnp.int32) // BLK)


def _route(xf, router_w, bias2d):
    return pl.pallas_call(
        _route_body,
        out_shape=[
            jax.ShapeDtypeStruct((SEQ, 1), jnp.int32),
            jax.ShapeDtypeStruct((SEQ, 1), jnp.int32),
            jax.ShapeDtypeStruct((SEQ, 1), jnp.float32),
            jax.ShapeDtypeStruct((SEQ, 1), jnp.float32),
            jax.ShapeDtypeStruct((NBLK, 1), jnp.int32),
            jax.ShapeDtypeStruct((1, 1), jnp.int32),
        ],
    )(xf, router_w, bias2d)


def _dispatch_body(x_hbm, p0_hbm, p1_hbm, xs_hbm, rows_v, i0_v, i1_v, sem):
    wid = lax.axis_index("s") * NC + lax.axis_index("c")
    base = wid * TPW

    def chunk(j, carry):
        b = base + j * SUB
        pltpu.sync_copy(x_hbm.at[pl.ds(b, SUB)], rows_v)
        pltpu.sync_copy(p0_hbm.at[pl.ds(b, SUB)], i0_v)
        pltpu.sync_copy(p1_hbm.at[pl.ds(b, SUB)], i1_v)
        c0 = pltpu.async_copy(rows_v, xs_hbm.at[i0_v], sem)
        c1 = pltpu.async_copy(rows_v, xs_hbm.at[i1_v], sem)
        c0.wait()
        c1.wait()
        return carry

    lax.fori_loop(0, TPW // SUB, chunk, 0)


@functools.cache
def _dispatch():
    return pl.kernel(
        _dispatch_body,
        out_type=jax.ShapeDtypeStruct((NPAD, DIM), jnp.float32),
        mesh=plsc.VectorSubcoreMesh(core_axis_name="c", subcore_axis_name="s",
                                    num_cores=NC, num_subcores=NS),
        scratch_types=[
            pltpu.VMEM((SUB, DIM), jnp.float32),
            pltpu.VMEM((SUB,), jnp.int32),
            pltpu.VMEM((SUB,), jnp.int32),
            pltpu.SemaphoreType.DMA,
        ],
    )


def _ff_block(xb, w1, w3, w2):
    a = lax.dot_general(xb, w1, (((1,), (1,)), ((), ())),
                        preferred_element_type=jnp.float32)
    g = lax.dot_general(xb, w3, (((1,), (1,)), ((), ())),
                        preferred_element_type=jnp.float32)
    h = (a * jax.nn.sigmoid(a)) * g
    return lax.dot_general(h, w2,
                           (((1,), (1,)), ((), ())),
                           preferred_element_type=jnp.float32)


def _gemm_body(bk_ref, xs_ref, w1_ref, w3_ref, w2_ref, o_ref):
    i = pl.program_id(0)

    @pl.when(i < bk_ref[NBLK])
    def _():
        o_ref[...] = _ff_block(xs_ref[...], w1_ref[0], w3_ref[0], w2_ref[0])


def _gemm(bk, xs, w1b, w3b, w2b):
    grid_spec = pltpu.PrefetchScalarGridSpec(
        num_scalar_prefetch=1,
        grid=(NBLK,),
        in_specs=[
            pl.BlockSpec((BLK, DIM), lambda i, bk: (i, 0)),
            pl.BlockSpec((1, HID, DIM), lambda i, bk: (bk[i], 0, 0)),
            pl.BlockSpec((1, HID, DIM), lambda i, bk: (bk[i], 0, 0)),
            pl.BlockSpec((1, DIM, HID), lambda i, bk: (bk[i], 0, 0)),
        ],
        out_specs=pl.BlockSpec((BLK, DIM), lambda i, bk: (i, 0)),
    )
    return pl.pallas_call(
        _gemm_body,
        grid_spec=grid_spec,
        out_shape=jax.ShapeDtypeStruct((NPAD, DIM), jnp.float32),
    )(bk, xs, w1b, w3b, w2b)


def _shared_body(x_ref, w1_ref, w3_ref, w2_ref, o_ref):
    o_ref[...] = _ff_block(x_ref[...], w1_ref[...], w3_ref[...], w2_ref[...])


def _shared(xf, sw1b, sw3b, sw2b):
    return pl.pallas_call(
        _shared_body,
        grid=(SEQ // SB,),
        in_specs=[
            pl.BlockSpec((SB, DIM), lambda i: (i, 0)),
            pl.BlockSpec((HID, DIM), lambda i: (0, 0)),
            pl.BlockSpec((HID, DIM), lambda i: (0, 0)),
            pl.BlockSpec((DIM, HID), lambda i: (0, 0)),
        ],
        out_specs=pl.BlockSpec((SB, DIM), lambda i: (i, 0)),
        out_shape=jax.ShapeDtypeStruct((SEQ, DIM), jnp.float32),
    )(xf, sw1b, sw3b, sw2b)


def _combine_body(ys_hbm, sh_hbm, p0_hbm, p1_hbm, w0_hbm, w1_hbm, out_hbm,
                  acc_v, r0_v, r1_v, w0_v, w1_v, i0_v, i1_v, sem):
    wid = lax.axis_index("s") * NC + lax.axis_index("c")
    base = wid * TPW

    def chunk(j, carry):
        b = base + j * SUB
        pltpu.sync_copy(sh_hbm.at[pl.ds(b, SUB)], acc_v)
        pltpu.sync_copy(p0_hbm.at[pl.ds(b, SUB)], i0_v)
        pltpu.sync_copy(p1_hbm.at[pl.ds(b, SUB)], i1_v)
        pltpu.sync_copy(w0_hbm.at[pl.ds(b, SUB)], w0_v)
        pltpu.sync_copy(w1_hbm.at[pl.ds(b, SUB)], w1_v)
        c0 = pltpu.async_copy(ys_hbm.at[i0_v], r0_v, sem)
        c1 = pltpu.async_copy(ys_hbm.at[i1_v], r1_v, sem)
        c0.wait()
        c1.wait()
        w0vec = w0_v[...]
        w1vec = w1_v[...]
        ws = [(w0vec[t], w1vec[t]) for t in range(SUB)]

        def col(c, carry2):
            s = pl.ds(c * 16, 16)
            for t in range(SUB):
                a, g = ws[t]
                acc_v[t, s] = acc_v[t, s] + a * r0_v[t, s] + g * r1_v[t, s]
            return carry2

        lax.fori_loop(0, DIM // 16, col, 0)
        pltpu.sync_copy(acc_v, out_hbm.at[pl.ds(b, SUB)])
        return carry

    lax.fori_loop(0, TPW // SUB, chunk, 0)


@functools.cache
def _combine():
    return pl.kernel(
        _combine_body,
        out_type=jax.ShapeDtypeStruct((SEQ, DIM), jnp.float32),
        mesh=plsc.VectorSubcoreMesh(core_axis_name="c", subcore_axis_name="s",
                                    num_cores=NC, num_subcores=NS),
        scratch_types=[
            pltpu.VMEM((SUB, DIM), jnp.float32),
            pltpu.VMEM((SUB, DIM), jnp.float32),
            pltpu.VMEM((SUB, DIM), jnp.float32),
            pltpu.VMEM((SUB,), jnp.float32),
            pltpu.VMEM((SUB,), jnp.float32),
            pltpu.VMEM((SUB,), jnp.int32),
            pltpu.VMEM((SUB,), jnp.int32),
            pltpu.SemaphoreType.DMA,
        ],
    )


@jax.jit
def kernel(x, router_w, W1, W3, W2, sw1, sw3, sw2, experts_bias):
    xf = x.reshape(SEQ, DIM)
    pos0, pos1, w0, w1, be, na = _route(xf, router_w,
                                        experts_bias.reshape(1, NE))
    bk = jnp.concatenate([be.reshape(NBLK), na.reshape(1)])
    p0 = pos0.reshape(SEQ)
    p1 = pos1.reshape(SEQ)
    xs = _dispatch()(xf, p0, p1)
    ys = _gemm(bk, xs, W1, W3, W2)
    sh = _shared(xf, sw1, sw3, sw2)
    out = _combine()(ys, sh, p0, p1, w0.reshape(SEQ), w1.reshape(SEQ))
    return out.reshape(x.shape)


# speedup vs baseline: 1.5842x; 1.1755x over previous
import functools

import jax
import jax.numpy as jnp
from jax import lax
from jax.experimental import pallas as pl
from jax.experimental.pallas import tpu as pltpu
from jax.experimental.pallas import tpu_sc as plsc

DIM = 2048
HID = 1024
NE = 8
SEQ = 2048
BLK = 256
NBLK = 2 * SEQ // BLK + NE
NPAD = NBLK * BLK
MAXB = SEQ // BLK
NC = 2
NS = 16
NW = NC * NS
TPW = SEQ // NW
SUB = 16
SB = 1024


def _route_body(x_ref, rw_ref, b_ref, pos0_ref, pos1_ref, w0_ref, w1_ref,
                bk_ref):
    x = x_ref[...]
    logits = lax.dot_general(x, rw_ref[...], (((1,), (1,)), ((), ())),
                             preferred_element_type=jnp.float32)
    logits = logits + b_ref[...]
    eidx = lax.broadcasted_iota(jnp.int32, (SEQ, NE), 1)
    m0 = jnp.max(logits, axis=1, keepdims=True)
    i0 = jnp.min(jnp.where(logits == m0, eidx, NE), axis=1, keepdims=True)
    sel0 = eidx == i0
    l2 = jnp.where(sel0, -jnp.inf, logits)
    m1 = jnp.max(l2, axis=1, keepdims=True)
    i1 = jnp.min(jnp.where(l2 == m1, eidx, NE), axis=1, keepdims=True)
    sel1 = eidx == i1
    d = jnp.exp(m1 - m0)
    inv = 1.0 / (1.0 + d)
    w0_ref[...] = inv
    w1_ref[...] = d * inv
    counts = sel0.astype(jnp.float32) + sel1.astype(jnp.float32)
    r_i = lax.broadcasted_iota(jnp.int32, (SEQ, SEQ), 0)
    c_i = lax.broadcasted_iota(jnp.int32, (SEQ, SEQ), 1)
    tri = (c_i < r_i).astype(jnp.float32)
    csum = lax.dot_general(tri, counts, (((1,), (0,)), ((), ())),
                           preferred_element_type=jnp.float32)
    tot = jnp.sum(counts, axis=0, keepdims=True)
    padded = jnp.ceil(tot / BLK) * BLK
    u_r = lax.broadcasted_iota(jnp.int32, (NE, NE), 0)
    u_c = lax.broadcasted_iota(jnp.int32, (NE, NE), 1)
    upper = (u_r < u_c).astype(jnp.float32)
    gbase = lax.dot_general(padded, upper, (((1,), (0,)), ((), ())),
                            preferred_element_type=jnp.float32)
    slot = gbase + csum
    pos0_ref[...] = jnp.sum(jnp.where(sel0, slot, 0.0), axis=1,
                            keepdims=True).astype(jnp.int32)
    pos1_ref[...] = jnp.sum(jnp.where(sel1, slot, 0.0), axis=1,
                            keepdims=True).astype(jnp.int32)
    sb = (gbase / BLK).astype(jnp.int32)
    nb = (padded / BLK).astype(jnp.int32)
    bk_ref[...] = jnp.concatenate([sb, nb], axis=1)


def _route(xf, router_w, bias2d):
    return pl.pallas_call(
        _route_body,
        out_shape=[
            jax.ShapeDtypeStruct((SEQ, 1), jnp.int32),
            jax.ShapeDtypeStruct((SEQ, 1), jnp.int32),
            jax.ShapeDtypeStruct((SEQ, 1), jnp.float32),
            jax.ShapeDtypeStruct((SEQ, 1), jnp.float32),
            jax.ShapeDtypeStruct((1, 2 * NE), jnp.int32),
        ],
    )(xf, router_w, bias2d)


def _dispatch_body(x_hbm, p0_hbm, p1_hbm, xs_hbm, rows_v, i0_v, i1_v, sem):
    wid = lax.axis_index("s") * NC + lax.axis_index("c")
    base = wid * TPW

    def chunk(j, carry):
        b = base + j * SUB
        pltpu.sync_copy(x_hbm.at[pl.ds(b, SUB)], rows_v)
        pltpu.sync_copy(p0_hbm.at[pl.ds(b, SUB)], i0_v)
        pltpu.sync_copy(p1_hbm.at[pl.ds(b, SUB)], i1_v)
        c0 = pltpu.async_copy(rows_v, xs_hbm.at[i0_v], sem)
        c1 = pltpu.async_copy(rows_v, xs_hbm.at[i1_v], sem)
        c0.wait()
        c1.wait()
        return carry

    lax.fori_loop(0, TPW // SUB, chunk, 0)


@functools.cache
def _dispatch():
    return pl.kernel(
        _dispatch_body,
        out_type=jax.ShapeDtypeStruct((NPAD, DIM), jnp.float32),
        mesh=plsc.VectorSubcoreMesh(core_axis_name="c", subcore_axis_name="s",
                                    num_cores=NC, num_subcores=NS),
        scratch_types=[
            pltpu.VMEM((SUB, DIM), jnp.float32),
            pltpu.VMEM((SUB,), jnp.int32),
            pltpu.VMEM((SUB,), jnp.int32),
            pltpu.SemaphoreType.DMA,
        ],
    )


def _ff_block(xb, w1, w3, w2):
    a = lax.dot_general(xb, w1, (((1,), (1,)), ((), ())),
                        preferred_element_type=jnp.float32)
    g = lax.dot_general(xb, w3, (((1,), (1,)), ((), ())),
                        preferred_element_type=jnp.float32)
    h = (a * jax.nn.sigmoid(a)) * g
    return lax.dot_general(h, w2,
                           (((1,), (1,)), ((), ())),
                           preferred_element_type=jnp.float32)


def _gemm_body(bk_ref, xs_ref, w1_ref, w3_ref, w2_ref, o_ref):
    e = pl.program_id(0)
    j = pl.program_id(1)

    @pl.when(j < bk_ref[NE + e])
    def _():
        o_ref[...] = _ff_block(xs_ref[...], w1_ref[0], w3_ref[0], w2_ref[0])


def _row_map(e, j, bk):
    idx = bk[e] + jnp.minimum(j, bk[NE + e] - 1)
    return (jnp.maximum(idx, 0), 0)


def _gemm(bk, xs, w1b, w3b, w2b):
    grid_spec = pltpu.PrefetchScalarGridSpec(
        num_scalar_prefetch=1,
        grid=(NE, MAXB),
        in_specs=[
            pl.BlockSpec((BLK, DIM), _row_map),
            pl.BlockSpec((1, HID, DIM), lambda e, j, bk: (e, 0, 0)),
            pl.BlockSpec((1, HID, DIM), lambda e, j, bk: (e, 0, 0)),
            pl.BlockSpec((1, DIM, HID), lambda e, j, bk: (e, 0, 0)),
        ],
        out_specs=pl.BlockSpec((BLK, DIM), _row_map),
    )
    return pl.pallas_call(
        _gemm_body,
        grid_spec=grid_spec,
        out_shape=jax.ShapeDtypeStruct((NPAD, DIM), jnp.float32),
        compiler_params=pltpu.CompilerParams(
            vmem_limit_bytes=100 * 1024 * 1024),
    )(bk, xs, w1b, w3b, w2b)


def _shared_body(x_ref, w1_ref, w3_ref, w2_ref, o_ref):
    o_ref[...] = _ff_block(x_ref[...], w1_ref[...], w3_ref[...], w2_ref[...])


def _shared(xf, sw1b, sw3b, sw2b):
    return pl.pallas_call(
        _shared_body,
        grid=(SEQ // SB,),
        in_specs=[
            pl.BlockSpec((SB, DIM), lambda i: (i, 0)),
            pl.BlockSpec((HID, DIM), lambda i: (0, 0)),
            pl.BlockSpec((HID, DIM), lambda i: (0, 0)),
            pl.BlockSpec((DIM, HID), lambda i: (0, 0)),
        ],
        out_specs=pl.BlockSpec((SB, DIM), lambda i: (i, 0)),
        out_shape=jax.ShapeDtypeStruct((SEQ, DIM), jnp.float32),
        compiler_params=pltpu.CompilerParams(
            vmem_limit_bytes=100 * 1024 * 1024),
    )(xf, sw1b, sw3b, sw2b)


def _combine_body(ys_hbm, sh_hbm, p0_hbm, p1_hbm, w0_hbm, w1_hbm, out_hbm,
                  acc_v, r0_v, r1_v, w0_v, w1_v, i0_v, i1_v, sem):
    wid = lax.axis_index("s") * NC + lax.axis_index("c")
    base = wid * TPW

    def chunk(j, carry):
        b = base + j * SUB
        pltpu.sync_copy(sh_hbm.at[pl.ds(b, SUB)], acc_v)
        pltpu.sync_copy(p0_hbm.at[pl.ds(b, SUB)], i0_v)
        pltpu.sync_copy(p1_hbm.at[pl.ds(b, SUB)], i1_v)
        pltpu.sync_copy(w0_hbm.at[pl.ds(b, SUB)], w0_v)
        pltpu.sync_copy(w1_hbm.at[pl.ds(b, SUB)], w1_v)
        c0 = pltpu.async_copy(ys_hbm.at[i0_v], r0_v, sem)
        c1 = pltpu.async_copy(ys_hbm.at[i1_v], r1_v, sem)
        c0.wait()
        c1.wait()
        w0vec = w0_v[...]
        w1vec = w1_v[...]
        ws = [(w0vec[t], w1vec[t]) for t in range(SUB)]

        def col(c, carry2):
            s = pl.ds(c * 16, 16)
            for t in range(SUB):
                a, g = ws[t]
                acc_v[t, s] = acc_v[t, s] + a * r0_v[t, s] + g * r1_v[t, s]
            return carry2

        lax.fori_loop(0, DIM // 16, col, 0)
        pltpu.sync_copy(acc_v, out_hbm.at[pl.ds(b, SUB)])
        return carry

    lax.fori_loop(0, TPW // SUB, chunk, 0)


@functools.cache
def _combine():
    return pl.kernel(
        _combine_body,
        out_type=jax.ShapeDtypeStruct((SEQ, DIM), jnp.float32),
        mesh=plsc.VectorSubcoreMesh(core_axis_name="c", subcore_axis_name="s",
                                    num_cores=NC, num_subcores=NS),
        scratch_types=[
            pltpu.VMEM((SUB, DIM), jnp.float32),
            pltpu.VMEM((SUB, DIM), jnp.float32),
            pltpu.VMEM((SUB, DIM), jnp.float32),
            pltpu.VMEM((SUB,), jnp.float32),
            pltpu.VMEM((SUB,), jnp.float32),
            pltpu.VMEM((SUB,), jnp.int32),
            pltpu.VMEM((SUB,), jnp.int32),
            pltpu.SemaphoreType.DMA,
        ],
    )


@jax.jit
def kernel(x, router_w, W1, W3, W2, sw1, sw3, sw2, experts_bias):
    xf = x.reshape(SEQ, DIM)
    pos0, pos1, w0, w1, bk2 = _route(xf, router_w,
                                     experts_bias.reshape(1, NE))
    bk = bk2.reshape(2 * NE)
    p0 = pos0.reshape(SEQ)
    p1 = pos1.reshape(SEQ)
    xs = _dispatch()(xf, p0, p1)
    ys = _gemm(bk, xs, W1, W3, W2)
    sh = _shared(xf, sw1, sw3, sw2)
    out = _combine()(ys, sh, p0, p1, w0.reshape(SEQ), w1.reshape(SEQ))
    return out.reshape(x.shape)


# speedup vs baseline: 1.6134x; 1.0184x over previous
import functools

import jax
import jax.numpy as jnp
from jax import lax
from jax.experimental import pallas as pl
from jax.experimental.pallas import tpu as pltpu
from jax.experimental.pallas import tpu_sc as plsc

DIM = 2048
HID = 1024
NE = 8
SEQ = 2048
BLK = 256
NBLK = 2 * SEQ // BLK + NE
NPAD = NBLK * BLK
MAXB = SEQ // BLK
NC = 2
NS = 16
NW = NC * NS
TPW = SEQ // NW
SUB = 16
SB = 1024


def _route_body(x_ref, rw_ref, b_ref, pos0_ref, pos1_ref, w0_ref, w1_ref,
                bk_ref):
    x = x_ref[...]
    logits = lax.dot_general(x, rw_ref[...], (((1,), (1,)), ((), ())),
                             preferred_element_type=jnp.float32)
    logits = logits + b_ref[...]
    eidx = lax.broadcasted_iota(jnp.int32, (SEQ, NE), 1)
    m0 = jnp.max(logits, axis=1, keepdims=True)
    i0 = jnp.min(jnp.where(logits == m0, eidx, NE), axis=1, keepdims=True)
    sel0 = eidx == i0
    l2 = jnp.where(sel0, -jnp.inf, logits)
    m1 = jnp.max(l2, axis=1, keepdims=True)
    i1 = jnp.min(jnp.where(l2 == m1, eidx, NE), axis=1, keepdims=True)
    sel1 = eidx == i1
    d = jnp.exp(m1 - m0)
    inv = 1.0 / (1.0 + d)
    w0_ref[...] = inv
    w1_ref[...] = d * inv
    counts = sel0.astype(jnp.float32) + sel1.astype(jnp.float32)
    r_i = lax.broadcasted_iota(jnp.int32, (SEQ, SEQ), 0)
    c_i = lax.broadcasted_iota(jnp.int32, (SEQ, SEQ), 1)
    tri = (c_i < r_i).astype(jnp.float32)
    csum = lax.dot_general(tri, counts, (((1,), (0,)), ((), ())),
                           preferred_element_type=jnp.float32)
    tot = jnp.sum(counts, axis=0, keepdims=True)
    padded = jnp.ceil(tot / BLK) * BLK
    u_r = lax.broadcasted_iota(jnp.int32, (NE, NE), 0)
    u_c = lax.broadcasted_iota(jnp.int32, (NE, NE), 1)
    upper = (u_r < u_c).astype(jnp.float32)
    gbase = lax.dot_general(padded, upper, (((1,), (0,)), ((), ())),
                            preferred_element_type=jnp.float32)
    slot = gbase + csum
    pos0_ref[...] = jnp.sum(jnp.where(sel0, slot, 0.0), axis=1,
                            keepdims=True).astype(jnp.int32)
    pos1_ref[...] = jnp.sum(jnp.where(sel1, slot, 0.0), axis=1,
                            keepdims=True).astype(jnp.int32)
    sb = (gbase / BLK).astype(jnp.int32)
    nb = (padded / BLK).astype(jnp.int32)
    bk_ref[...] = jnp.concatenate([sb, nb], axis=1)


def _route(xf, router_w, bias2d):
    return pl.pallas_call(
        _route_body,
        out_shape=[
            jax.ShapeDtypeStruct((SEQ, 1), jnp.int32),
            jax.ShapeDtypeStruct((SEQ, 1), jnp.int32),
            jax.ShapeDtypeStruct((SEQ, 1), jnp.float32),
            jax.ShapeDtypeStruct((SEQ, 1), jnp.float32),
            jax.ShapeDtypeStruct((1, 2 * NE), jnp.int32),
        ],
    )(xf, router_w, bias2d)


def _dispatch_body(x_hbm, p0_hbm, p1_hbm, xs_hbm,
                   rows_a, rows_b, i0_a, i0_b, i1_a, i1_b,
                   lsem_a, lsem_b, ssem_a, ssem_b):
    wid = lax.axis_index("s") * NC + lax.axis_index("c")
    base = wid * TPW
    nch = TPW // SUB
    rows = (rows_a, rows_b)
    i0s = (i0_a, i0_b)
    i1s = (i1_a, i1_b)
    lsems = (lsem_a, lsem_b)
    ssems = (ssem_a, ssem_b)

    def load(g):
        b = base + g * SUB
        s = g % 2
        pltpu.sync_copy(p0_hbm.at[pl.ds(b, SUB)], i0s[s])
        pltpu.sync_copy(p1_hbm.at[pl.ds(b, SUB)], i1s[s])
        return pltpu.async_copy(x_hbm.at[pl.ds(b, SUB)], rows[s], lsems[s])

    loads = {0: load(0)}
    scats = {}
    for g in range(nch):
        s = g % 2
        if g + 1 < nch:
            if g >= 1:
                for c in scats[g - 1]:
                    c.wait()
            loads[g + 1] = load(g + 1)
        loads[g].wait()
        scats[g] = (pltpu.async_copy(rows[s], xs_hbm.at[i0s[s]], ssems[s]),
                    pltpu.async_copy(rows[s], xs_hbm.at[i1s[s]], ssems[s]))
    for g in (nch - 2, nch - 1):
        for c in scats[g]:
            c.wait()


@functools.cache
def _dispatch():
    return pl.kernel(
        _dispatch_body,
        out_type=jax.ShapeDtypeStruct((NPAD, DIM), jnp.float32),
        mesh=plsc.VectorSubcoreMesh(core_axis_name="c", subcore_axis_name="s",
                                    num_cores=NC, num_subcores=NS),
        scratch_types=[
            pltpu.VMEM((SUB, DIM), jnp.float32),
            pltpu.VMEM((SUB, DIM), jnp.float32),
            pltpu.VMEM((SUB,), jnp.int32),
            pltpu.VMEM((SUB,), jnp.int32),
            pltpu.VMEM((SUB,), jnp.int32),
            pltpu.VMEM((SUB,), jnp.int32),
            pltpu.SemaphoreType.DMA,
            pltpu.SemaphoreType.DMA,
            pltpu.SemaphoreType.DMA,
            pltpu.SemaphoreType.DMA,
        ],
    )


def _ff_block(xb, w1, w3, w2):
    a = lax.dot_general(xb, w1, (((1,), (1,)), ((), ())),
                        preferred_element_type=jnp.float32)
    g = lax.dot_general(xb, w3, (((1,), (1,)), ((), ())),
                        preferred_element_type=jnp.float32)
    h = (a * jax.nn.sigmoid(a)) * g
    return lax.dot_general(h, w2,
                           (((1,), (1,)), ((), ())),
                           preferred_element_type=jnp.float32)


def _gemm_body(bk_ref, xs_ref, w1_ref, w3_ref, w2_ref, o_ref):
    e = pl.program_id(0)
    j = pl.program_id(1)

    @pl.when(j < bk_ref[NE + e])
    def _():
        o_ref[...] = _ff_block(xs_ref[...], w1_ref[0], w3_ref[0], w2_ref[0])


def _row_map(e, j, bk):
    idx = bk[e] + jnp.minimum(j, bk[NE + e] - 1)
    return (jnp.maximum(idx, 0), 0)


def _gemm(bk, xs, w1b, w3b, w2b):
    grid_spec = pltpu.PrefetchScalarGridSpec(
        num_scalar_prefetch=1,
        grid=(NE, MAXB),
        in_specs=[
            pl.BlockSpec((BLK, DIM), _row_map),
            pl.BlockSpec((1, HID, DIM), lambda e, j, bk: (e, 0, 0)),
            pl.BlockSpec((1, HID, DIM), lambda e, j, bk: (e, 0, 0)),
            pl.BlockSpec((1, DIM, HID), lambda e, j, bk: (e, 0, 0)),
        ],
        out_specs=pl.BlockSpec((BLK, DIM), _row_map),
    )
    return pl.pallas_call(
        _gemm_body,
        grid_spec=grid_spec,
        out_shape=jax.ShapeDtypeStruct((NPAD, DIM), jnp.float32),
        compiler_params=pltpu.CompilerParams(
            vmem_limit_bytes=100 * 1024 * 1024),
    )(bk, xs, w1b, w3b, w2b)


def _shared_body(x_ref, w1_ref, w3_ref, w2_ref, o_ref):
    o_ref[...] = _ff_block(x_ref[...], w1_ref[...], w3_ref[...], w2_ref[...])


def _shared(xf, sw1b, sw3b, sw2b):
    return pl.pallas_call(
        _shared_body,
        grid=(SEQ // SB,),
        in_specs=[
            pl.BlockSpec((SB, DIM), lambda i: (i, 0)),
            pl.BlockSpec((HID, DIM), lambda i: (0, 0)),
            pl.BlockSpec((HID, DIM), lambda i: (0, 0)),
            pl.BlockSpec((DIM, HID), lambda i: (0, 0)),
        ],
        out_specs=pl.BlockSpec((SB, DIM), lambda i: (i, 0)),
        out_shape=jax.ShapeDtypeStruct((SEQ, DIM), jnp.float32),
        compiler_params=pltpu.CompilerParams(
            vmem_limit_bytes=100 * 1024 * 1024),
    )(xf, sw1b, sw3b, sw2b)


SUBC = 8


def _combine_body(ys_hbm, sh_hbm, p0_hbm, p1_hbm, w0_hbm, w1_hbm, out_hbm,
                  acc_a, acc_b, r0_a, r0_b, r1_a, r1_b,
                  w0_a, w0_b, w1_a, w1_b, i0_a, i0_b, i1_a, i1_b,
                  lsem_a, lsem_b, osem_a, osem_b):
    wid = lax.axis_index("s") * NC + lax.axis_index("c")
    base = wid * TPW
    nch = TPW // SUBC
    accs = (acc_a, acc_b)
    r0s = (r0_a, r0_b)
    r1s = (r1_a, r1_b)
    w0s = (w0_a, w0_b)
    w1s = (w1_a, w1_b)
    i0s = (i0_a, i0_b)
    i1s = (i1_a, i1_b)
    lsems = (lsem_a, lsem_b)
    osems = (osem_a, osem_b)

    def load(g):
        b = base + g * SUBC
        s = g % 2
        if g % 2 == 0:
            p = (g // 2) % 2
            pltpu.sync_copy(w0_hbm.at[pl.ds(b, 2 * SUBC)], w0s[p])
            pltpu.sync_copy(w1_hbm.at[pl.ds(b, 2 * SUBC)], w1s[p])
        pltpu.sync_copy(p0_hbm.at[pl.ds(b, SUBC)], i0s[s])
        pltpu.sync_copy(p1_hbm.at[pl.ds(b, SUBC)], i1s[s])
        return (pltpu.async_copy(sh_hbm.at[pl.ds(b, SUBC)], accs[s], lsems[s]),
                pltpu.async_copy(ys_hbm.at[i0s[s]], r0s[s], lsems[s]),
                pltpu.async_copy(ys_hbm.at[i1s[s]], r1s[s], lsems[s]))

    loads = {0: load(0)}
    outs = {}
    for g in range(nch):
        s = g % 2
        if g + 1 < nch:
            if g >= 1:
                outs[g - 1].wait()
            loads[g + 1] = load(g + 1)
        for c in loads[g]:
            c.wait()
        w0vec = w0s[(g // 2) % 2][...]
        w1vec = w1s[(g // 2) % 2][...]
        off = (g % 2) * SUBC
        ws = [(w0vec[off + t], w1vec[off + t]) for t in range(SUBC)]
        acc, r0, r1 = accs[s], r0s[s], r1s[s]

        def col(c, carry, ws=ws, acc=acc, r0=r0, r1=r1):
            sl = pl.ds(c * 16, 16)
            for t in range(SUBC):
                a, gg = ws[t]
                acc[t, sl] = acc[t, sl] + a * r0[t, sl] + gg * r1[t, sl]
            return carry

        lax.fori_loop(0, DIM // 16, col, 0)
        outs[g] = pltpu.async_copy(acc, out_hbm.at[pl.ds(base + g * SUBC, SUBC)],
                                   osems[s])
    outs[nch - 2].wait()
    outs[nch - 1].wait()


@functools.cache
def _combine():
    return pl.kernel(
        _combine_body,
        out_type=jax.ShapeDtypeStruct((SEQ, DIM), jnp.float32),
        mesh=plsc.VectorSubcoreMesh(core_axis_name="c", subcore_axis_name="s",
                                    num_cores=NC, num_subcores=NS),
        scratch_types=[
            pltpu.VMEM((SUBC, DIM), jnp.float32),
            pltpu.VMEM((SUBC, DIM), jnp.float32),
            pltpu.VMEM((SUBC, DIM), jnp.float32),
            pltpu.VMEM((SUBC, DIM), jnp.float32),
            pltpu.VMEM((SUBC, DIM), jnp.float32),
            pltpu.VMEM((SUBC, DIM), jnp.float32),
            pltpu.VMEM((16,), jnp.float32),
            pltpu.VMEM((16,), jnp.float32),
            pltpu.VMEM((16,), jnp.float32),
            pltpu.VMEM((16,), jnp.float32),
            pltpu.VMEM((SUBC,), jnp.int32),
            pltpu.VMEM((SUBC,), jnp.int32),
            pltpu.VMEM((SUBC,), jnp.int32),
            pltpu.VMEM((SUBC,), jnp.int32),
            pltpu.SemaphoreType.DMA,
            pltpu.SemaphoreType.DMA,
            pltpu.SemaphoreType.DMA,
            pltpu.SemaphoreType.DMA,
        ],
    )


@jax.jit
def kernel(x, router_w, W1, W3, W2, sw1, sw3, sw2, experts_bias):
    xf = x.reshape(SEQ, DIM)
    pos0, pos1, w0, w1, bk2 = _route(xf, router_w,
                                     experts_bias.reshape(1, NE))
    bk = bk2.reshape(2 * NE)
    p0 = pos0.reshape(SEQ)
    p1 = pos1.reshape(SEQ)
    xs = _dispatch()(xf, p0, p1)
    ys = _gemm(bk, xs, W1, W3, W2)
    sh = _shared(xf, sw1, sw3, sw2)
    out = _combine()(ys, sh, p0, p1, w0.reshape(SEQ), w1.reshape(SEQ))
    return out.reshape(x.shape)


# speedup vs baseline: 1.8610x; 1.1534x over previous
import functools

import jax
import jax.numpy as jnp
from jax import lax
from jax.experimental import pallas as pl
from jax.experimental.pallas import tpu as pltpu
from jax.experimental.pallas import tpu_sc as plsc

DIM = 2048
HID = 1024
NE = 8
SEQ = 2048
BLK = 256
NBLK = 2 * SEQ // BLK + NE
NPAD = NBLK * BLK
MAXB = SEQ // BLK
NC = 2
NS = 16
NW = NC * NS
TPW = SEQ // NW
SUB = 16
SB = 1024


def _route_body(x_ref, rw_ref, b_ref, pos0_ref, pos1_ref, w0_ref, w1_ref,
                bk_ref):
    x = x_ref[...]
    logits = lax.dot_general(x, rw_ref[...], (((1,), (1,)), ((), ())),
                             preferred_element_type=jnp.float32)
    logits = logits + b_ref[...]
    eidx = lax.broadcasted_iota(jnp.int32, (SEQ, NE), 1)
    m0 = jnp.max(logits, axis=1, keepdims=True)
    i0 = jnp.min(jnp.where(logits == m0, eidx, NE), axis=1, keepdims=True)
    sel0 = eidx == i0
    l2 = jnp.where(sel0, -jnp.inf, logits)
    m1 = jnp.max(l2, axis=1, keepdims=True)
    i1 = jnp.min(jnp.where(l2 == m1, eidx, NE), axis=1, keepdims=True)
    sel1 = eidx == i1
    d = jnp.exp(m1 - m0)
    inv = 1.0 / (1.0 + d)
    w0_ref[...] = inv
    w1_ref[...] = d * inv
    counts = sel0.astype(jnp.float32) + sel1.astype(jnp.float32)
    r_i = lax.broadcasted_iota(jnp.int32, (SEQ, SEQ), 0)
    c_i = lax.broadcasted_iota(jnp.int32, (SEQ, SEQ), 1)
    tri = (c_i < r_i).astype(jnp.float32)
    csum = lax.dot_general(tri, counts, (((1,), (0,)), ((), ())),
                           preferred_element_type=jnp.float32)
    tot = jnp.sum(counts, axis=0, keepdims=True)
    padded = jnp.ceil(tot / BLK) * BLK
    u_r = lax.broadcasted_iota(jnp.int32, (NE, NE), 0)
    u_c = lax.broadcasted_iota(jnp.int32, (NE, NE), 1)
    upper = (u_r < u_c).astype(jnp.float32)
    gbase = lax.dot_general(padded, upper, (((1,), (0,)), ((), ())),
                            preferred_element_type=jnp.float32)
    slot = gbase + csum
    pos0_ref[...] = jnp.sum(jnp.where(sel0, slot, 0.0), axis=1,
                            keepdims=True).astype(jnp.int32)
    pos1_ref[...] = jnp.sum(jnp.where(sel1, slot, 0.0), axis=1,
                            keepdims=True).astype(jnp.int32)
    sb = (gbase / BLK).astype(jnp.int32)
    nb = (padded / BLK).astype(jnp.int32)
    bk_ref[...] = jnp.concatenate([sb, nb], axis=1)


def _route(xf, router_w, bias2d):
    return pl.pallas_call(
        _route_body,
        out_shape=[
            jax.ShapeDtypeStruct((SEQ, 1), jnp.int32),
            jax.ShapeDtypeStruct((SEQ, 1), jnp.int32),
            jax.ShapeDtypeStruct((SEQ, 1), jnp.float32),
            jax.ShapeDtypeStruct((SEQ, 1), jnp.float32),
            jax.ShapeDtypeStruct((1, 2 * NE), jnp.int32),
        ],
    )(xf, router_w, bias2d)


def _dispatch_body(x_hbm, p0_hbm, p1_hbm, xs_hbm,
                   rows_a, rows_b, i0_a, i0_b, i1_a, i1_b,
                   lsem_a, lsem_b, ssem_a, ssem_b):
    wid = lax.axis_index("s") * NC + lax.axis_index("c")
    base = wid * TPW
    nch = TPW // SUB
    rows = (rows_a, rows_b)
    i0s = (i0_a, i0_b)
    i1s = (i1_a, i1_b)
    lsems = (lsem_a, lsem_b)
    ssems = (ssem_a, ssem_b)

    def load(g):
        b = base + g * SUB
        s = g % 2
        pltpu.sync_copy(p0_hbm.at[pl.ds(b, SUB)], i0s[s])
        pltpu.sync_copy(p1_hbm.at[pl.ds(b, SUB)], i1s[s])
        return pltpu.async_copy(x_hbm.at[pl.ds(b, SUB)], rows[s], lsems[s])

    loads = {0: load(0)}
    scats = {}
    for g in range(nch):
        s = g % 2
        if g + 1 < nch:
            if g >= 1:
                for c in scats[g - 1]:
                    c.wait()
            loads[g + 1] = load(g + 1)
        loads[g].wait()
        scats[g] = (pltpu.async_copy(rows[s], xs_hbm.at[i0s[s]], ssems[s]),
                    pltpu.async_copy(rows[s], xs_hbm.at[i1s[s]], ssems[s]))
    for g in (nch - 2, nch - 1):
        for c in scats[g]:
            c.wait()


@functools.cache
def _dispatch():
    return pl.kernel(
        _dispatch_body,
        out_type=jax.ShapeDtypeStruct((NPAD, DIM), jnp.float32),
        mesh=plsc.VectorSubcoreMesh(core_axis_name="c", subcore_axis_name="s",
                                    num_cores=NC, num_subcores=NS),
        scratch_types=[
            pltpu.VMEM((SUB, DIM), jnp.float32),
            pltpu.VMEM((SUB, DIM), jnp.float32),
            pltpu.VMEM((SUB,), jnp.int32),
            pltpu.VMEM((SUB,), jnp.int32),
            pltpu.VMEM((SUB,), jnp.int32),
            pltpu.VMEM((SUB,), jnp.int32),
            pltpu.SemaphoreType.DMA,
            pltpu.SemaphoreType.DMA,
            pltpu.SemaphoreType.DMA,
            pltpu.SemaphoreType.DMA,
        ],
    )


def _ff_block(xb, w1, w3, w2):
    a = lax.dot_general(xb, w1, (((1,), (1,)), ((), ())),
                        preferred_element_type=jnp.float32)
    g = lax.dot_general(xb, w3, (((1,), (1,)), ((), ())),
                        preferred_element_type=jnp.float32)
    h = (a * jax.nn.sigmoid(a)) * g
    return lax.dot_general(h, w2,
                           (((1,), (1,)), ((), ())),
                           preferred_element_type=jnp.float32)


def _w_copies(w1_hbm, w3_hbm, w2_hbm, w1b, w3b, w2b, sem, e, s):
    return (pltpu.make_async_copy(w1_hbm.at[e], w1b.at[s], sem),
            pltpu.make_async_copy(w3_hbm.at[e], w3b.at[s], sem),
            pltpu.make_async_copy(w2_hbm.at[e], w2b.at[s], sem))


def _gemm_body(bk_ref, xs_ref, w1_hbm, w3_hbm, w2_hbm, o_ref,
               w1b, w3b, w2b, sem):
    e = pl.program_id(0)
    j = pl.program_id(1)

    @pl.when(j == 0)
    def _():
        @pl.when(e == 0)
        def _():
            for c in _w_copies(w1_hbm, w3_hbm, w2_hbm, w1b, w3b, w2b,
                               sem, 0, 0):
                c.start()

        @pl.when(e + 1 < NE)
        def _():
            for c in _w_copies(w1_hbm, w3_hbm, w2_hbm, w1b, w3b, w2b,
                               sem, e + 1, (e + 1) % 2):
                c.start()

        for c in _w_copies(w1_hbm, w3_hbm, w2_hbm, w1b, w3b, w2b,
                           sem, e, e % 2):
            c.wait()

    @pl.when(j < bk_ref[NE + e])
    def _():
        s = e % 2
        o_ref[...] = _ff_block(xs_ref[...], w1b[s], w3b[s], w2b[s])


def _row_map(e, j, bk):
    idx = bk[e] + jnp.minimum(j, bk[NE + e] - 1)
    return (jnp.maximum(idx, 0), 0)


def _gemm(bk, xs, w1b, w3b, w2b):
    grid_spec = pltpu.PrefetchScalarGridSpec(
        num_scalar_prefetch=1,
        grid=(NE, MAXB),
        in_specs=[
            pl.BlockSpec((BLK, DIM), _row_map),
            pl.BlockSpec(memory_space=pl.ANY),
            pl.BlockSpec(memory_space=pl.ANY),
            pl.BlockSpec(memory_space=pl.ANY),
        ],
        out_specs=pl.BlockSpec((BLK, DIM), _row_map),
        scratch_shapes=[
            pltpu.VMEM((2, HID, DIM), jnp.float32),
            pltpu.VMEM((2, HID, DIM), jnp.float32),
            pltpu.VMEM((2, DIM, HID), jnp.float32),
            pltpu.SemaphoreType.DMA,
        ],
    )
    return pl.pallas_call(
        _gemm_body,
        grid_spec=grid_spec,
        out_shape=jax.ShapeDtypeStruct((NPAD, DIM), jnp.float32),
        compiler_params=pltpu.CompilerParams(
            vmem_limit_bytes=100 * 1024 * 1024),
    )(bk, xs, w1b, w3b, w2b)


def _shared_body(x_ref, w1_ref, w3_ref, w2_ref, o_ref):
    o_ref[...] = _ff_block(x_ref[...], w1_ref[...], w3_ref[...], w2_ref[...])


def _shared(xf, sw1b, sw3b, sw2b):
    return pl.pallas_call(
        _shared_body,
        grid=(SEQ // SB,),
        in_specs=[
            pl.BlockSpec((SB, DIM), lambda i: (i, 0)),
            pl.BlockSpec((HID, DIM), lambda i: (0, 0)),
            pl.BlockSpec((HID, DIM), lambda i: (0, 0)),
            pl.BlockSpec((DIM, HID), lambda i: (0, 0)),
        ],
        out_specs=pl.BlockSpec((SB, DIM), lambda i: (i, 0)),
        out_shape=jax.ShapeDtypeStruct((SEQ, DIM), jnp.float32),
        compiler_params=pltpu.CompilerParams(
            vmem_limit_bytes=100 * 1024 * 1024),
    )(xf, sw1b, sw3b, sw2b)


SUBC = 8


def _combine_body(ys_hbm, sh_hbm, p0_hbm, p1_hbm, w0_hbm, w1_hbm, out_hbm,
                  acc_a, acc_b, r0_a, r0_b, r1_a, r1_b,
                  w0_a, w0_b, w1_a, w1_b, i0_a, i0_b, i1_a, i1_b,
                  lsem_a, lsem_b, osem_a, osem_b):
    wid = lax.axis_index("s") * NC + lax.axis_index("c")
    base = wid * TPW
    nch = TPW // SUBC
    accs = (acc_a, acc_b)
    r0s = (r0_a, r0_b)
    r1s = (r1_a, r1_b)
    w0s = (w0_a, w0_b)
    w1s = (w1_a, w1_b)
    i0s = (i0_a, i0_b)
    i1s = (i1_a, i1_b)
    lsems = (lsem_a, lsem_b)
    osems = (osem_a, osem_b)

    def load(g):
        b = base + g * SUBC
        s = g % 2
        if g % 2 == 0:
            p = (g // 2) % 2
            pltpu.sync_copy(w0_hbm.at[pl.ds(b, 2 * SUBC)], w0s[p])
            pltpu.sync_copy(w1_hbm.at[pl.ds(b, 2 * SUBC)], w1s[p])
        pltpu.sync_copy(p0_hbm.at[pl.ds(b, SUBC)], i0s[s])
        pltpu.sync_copy(p1_hbm.at[pl.ds(b, SUBC)], i1s[s])
        return (pltpu.async_copy(sh_hbm.at[pl.ds(b, SUBC)], accs[s], lsems[s]),
                pltpu.async_copy(ys_hbm.at[i0s[s]], r0s[s], lsems[s]),
                pltpu.async_copy(ys_hbm.at[i1s[s]], r1s[s], lsems[s]))

    loads = {0: load(0)}
    outs = {}
    for g in range(nch):
        s = g % 2
        if g + 1 < nch:
            if g >= 1:
                outs[g - 1].wait()
            loads[g + 1] = load(g + 1)
        for c in loads[g]:
            c.wait()
        w0vec = w0s[(g // 2) % 2][...]
        w1vec = w1s[(g // 2) % 2][...]
        off = (g % 2) * SUBC
        ws = [(w0vec[off + t], w1vec[off + t]) for t in range(SUBC)]
        acc, r0, r1 = accs[s], r0s[s], r1s[s]

        def col(c, carry, ws=ws, acc=acc, r0=r0, r1=r1):
            sl = pl.ds(c * 16, 16)
            for t in range(SUBC):
                a, gg = ws[t]
                acc[t, sl] = acc[t, sl] + a * r0[t, sl] + gg * r1[t, sl]
            return carry

        lax.fori_loop(0, DIM // 16, col, 0)
        outs[g] = pltpu.async_copy(acc, out_hbm.at[pl.ds(base + g * SUBC, SUBC)],
                                   osems[s])
    outs[nch - 2].wait()
    outs[nch - 1].wait()


@functools.cache
def _combine():
    return pl.kernel(
        _combine_body,
        out_type=jax.ShapeDtypeStruct((SEQ, DIM), jnp.float32),
        mesh=plsc.VectorSubcoreMesh(core_axis_name="c", subcore_axis_name="s",
                                    num_cores=NC, num_subcores=NS),
        scratch_types=[
            pltpu.VMEM((SUBC, DIM), jnp.float32),
            pltpu.VMEM((SUBC, DIM), jnp.float32),
            pltpu.VMEM((SUBC, DIM), jnp.float32),
            pltpu.VMEM((SUBC, DIM), jnp.float32),
            pltpu.VMEM((SUBC, DIM), jnp.float32),
            pltpu.VMEM((SUBC, DIM), jnp.float32),
            pltpu.VMEM((16,), jnp.float32),
            pltpu.VMEM((16,), jnp.float32),
            pltpu.VMEM((16,), jnp.float32),
            pltpu.VMEM((16,), jnp.float32),
            pltpu.VMEM((SUBC,), jnp.int32),
            pltpu.VMEM((SUBC,), jnp.int32),
            pltpu.VMEM((SUBC,), jnp.int32),
            pltpu.VMEM((SUBC,), jnp.int32),
            pltpu.SemaphoreType.DMA,
            pltpu.SemaphoreType.DMA,
            pltpu.SemaphoreType.DMA,
            pltpu.SemaphoreType.DMA,
        ],
    )


@jax.jit
def kernel(x, router_w, W1, W3, W2, sw1, sw3, sw2, experts_bias):
    xf = x.reshape(SEQ, DIM)
    pos0, pos1, w0, w1, bk2 = _route(xf, router_w,
                                     experts_bias.reshape(1, NE))
    bk = bk2.reshape(2 * NE)
    p0 = pos0.reshape(SEQ)
    p1 = pos1.reshape(SEQ)
    xs = _dispatch()(xf, p0, p1)
    ys = _gemm(bk, xs, W1, W3, W2)
    sh = _shared(xf, sw1, sw3, sw2)
    out = _combine()(ys, sh, p0, p1, w0.reshape(SEQ), w1.reshape(SEQ))
    return out.reshape(x.shape)


# speedup vs baseline: 1.8827x; 1.0117x over previous
import functools

import jax
import jax.numpy as jnp
from jax import lax
from jax.experimental import pallas as pl
from jax.experimental.pallas import tpu as pltpu
from jax.experimental.pallas import tpu_sc as plsc

DIM = 2048
HID = 1024
NE = 8
SEQ = 2048
BLK = 256
NBLK = 2 * SEQ // BLK + NE
NPAD = NBLK * BLK
MAXB = SEQ // BLK
LEAD = 4
NC = 2
NS = 16
NW = NC * NS
TPW = SEQ // NW
SUB = 16
SB = 1024


def _route_body(x_ref, rw_ref, b_ref, pos0_ref, pos1_ref, w0_ref, w1_ref,
                bk_ref):
    x = x_ref[...]
    logits = lax.dot_general(x, rw_ref[...], (((1,), (1,)), ((), ())),
                             preferred_element_type=jnp.float32)
    logits = logits + b_ref[...]
    eidx = lax.broadcasted_iota(jnp.int32, (SEQ, NE), 1)
    m0 = jnp.max(logits, axis=1, keepdims=True)
    i0 = jnp.min(jnp.where(logits == m0, eidx, NE), axis=1, keepdims=True)
    sel0 = eidx == i0
    l2 = jnp.where(sel0, -jnp.inf, logits)
    m1 = jnp.max(l2, axis=1, keepdims=True)
    i1 = jnp.min(jnp.where(l2 == m1, eidx, NE), axis=1, keepdims=True)
    sel1 = eidx == i1
    d = jnp.exp(m1 - m0)
    inv = 1.0 / (1.0 + d)
    w0_ref[...] = inv
    w1_ref[...] = d * inv
    counts = sel0.astype(jnp.float32) + sel1.astype(jnp.float32)
    r_i = lax.broadcasted_iota(jnp.int32, (SEQ, SEQ), 0)
    c_i = lax.broadcasted_iota(jnp.int32, (SEQ, SEQ), 1)
    tri = (c_i < r_i).astype(jnp.float32)
    csum = lax.dot_general(tri, counts, (((1,), (0,)), ((), ())),
                           preferred_element_type=jnp.float32)
    tot = jnp.sum(counts, axis=0, keepdims=True)
    padded = jnp.ceil(tot / BLK) * BLK
    u_r = lax.broadcasted_iota(jnp.int32, (NE, NE), 0)
    u_c = lax.broadcasted_iota(jnp.int32, (NE, NE), 1)
    upper = (u_r < u_c).astype(jnp.float32)
    gbase = lax.dot_general(padded, upper, (((1,), (0,)), ((), ())),
                            preferred_element_type=jnp.float32)
    slot = gbase + csum
    pos0_ref[...] = jnp.sum(jnp.where(sel0, slot, 0.0), axis=1,
                            keepdims=True).astype(jnp.int32)
    pos1_ref[...] = jnp.sum(jnp.where(sel1, slot, 0.0), axis=1,
                            keepdims=True).astype(jnp.int32)
    cum_in = gbase + padded
    start = (gbase / BLK).astype(jnp.int32)
    nonempty = padded > 0
    ordv = lax.dot_general(nonempty.astype(jnp.float32), upper,
                           (((1,), (0,)), ((), ())),
                           preferred_element_type=jnp.float32)
    slot_e = ordv.astype(jnp.int32) % 2
    na = jnp.sum(padded).astype(jnp.int32) // BLK
    t_list = []
    prev_t = jnp.int32(-1)
    for e in range(NE):
        s_e = start[0, e]
        p_e = jnp.where(e > 0, start[0, max(e - 1, 0)], 0)
        ne_e = padded[0, e] > 0
        t_e = jnp.maximum(jnp.maximum(s_e - LEAD, p_e), prev_t + 1)
        t_e = jnp.where(ne_e, t_e, jnp.int32(-1000000))
        prev_t = jnp.where(ne_e, t_e, prev_t)
        t_list.append(t_e)
    bidx = lax.broadcasted_iota(jnp.int32, (NBLK, NE), 0)
    eids = lax.broadcasted_iota(jnp.int32, (NBLK, NE), 1)
    t_mat = jnp.concatenate(
        [jnp.full((NBLK, 1), t, jnp.int32) for t in t_list], axis=1)
    hit = (bidx == t_mat).astype(jnp.int32)
    fse = jnp.sum(hit * (eids + 1), axis=1, keepdims=True) - 1
    fss = jnp.sum(hit * slot_e, axis=1, keepdims=True)
    cum_in_i = cum_in.astype(jnp.int32)
    bpos = bidx * BLK
    be = jnp.sum((bpos >= cum_in_i).astype(jnp.int32), axis=1, keepdims=True)
    sl = jnp.sum(jnp.where(eids == be, slot_e, 0), axis=1, keepdims=True)
    wt = jnp.sum(((bidx == start) & nonempty).astype(jnp.int32),
                 axis=1, keepdims=True)
    nacol = jnp.full((NBLK, 1), 0, jnp.int32) + na
    pad3 = jnp.full((NBLK, 3), 0, jnp.int32)
    bk_ref[...] = jnp.concatenate([sl, fse, fss, wt, nacol, pad3], axis=1)


def _route(xf, router_w, bias2d):
    return pl.pallas_call(
        _route_body,
        out_shape=[
            jax.ShapeDtypeStruct((SEQ, 1), jnp.int32),
            jax.ShapeDtypeStruct((SEQ, 1), jnp.int32),
            jax.ShapeDtypeStruct((SEQ, 1), jnp.float32),
            jax.ShapeDtypeStruct((SEQ, 1), jnp.float32),
            jax.ShapeDtypeStruct((NBLK, 8), jnp.int32),
        ],
    )(xf, router_w, bias2d)


def _dispatch_body(x_hbm, p0_hbm, p1_hbm, xs_hbm,
                   rows_a, rows_b, i0_a, i0_b, i1_a, i1_b,
                   lsem_a, lsem_b, ssem_a, ssem_b):
    wid = lax.axis_index("s") * NC + lax.axis_index("c")
    base = wid * TPW
    nch = TPW // SUB
    rows = (rows_a, rows_b)
    i0s = (i0_a, i0_b)
    i1s = (i1_a, i1_b)
    lsems = (lsem_a, lsem_b)
    ssems = (ssem_a, ssem_b)

    def load(g):
        b = base + g * SUB
        s = g % 2
        pltpu.sync_copy(p0_hbm.at[pl.ds(b, SUB)], i0s[s])
        pltpu.sync_copy(p1_hbm.at[pl.ds(b, SUB)], i1s[s])
        return pltpu.async_copy(x_hbm.at[pl.ds(b, SUB)], rows[s], lsems[s])

    loads = {0: load(0)}
    scats = {}
    for g in range(nch):
        s = g % 2
        if g + 1 < nch:
            if g >= 1:
                for c in scats[g - 1]:
                    c.wait()
            loads[g + 1] = load(g + 1)
        loads[g].wait()
        scats[g] = (pltpu.async_copy(rows[s], xs_hbm.at[i0s[s]], ssems[s]),
                    pltpu.async_copy(rows[s], xs_hbm.at[i1s[s]], ssems[s]))
    for g in (nch - 2, nch - 1):
        for c in scats[g]:
            c.wait()


@functools.cache
def _dispatch():
    return pl.kernel(
        _dispatch_body,
        out_type=jax.ShapeDtypeStruct((NPAD, DIM), jnp.float32),
        mesh=plsc.VectorSubcoreMesh(core_axis_name="c", subcore_axis_name="s",
                                    num_cores=NC, num_subcores=NS),
        scratch_types=[
            pltpu.VMEM((SUB, DIM), jnp.float32),
            pltpu.VMEM((SUB, DIM), jnp.float32),
            pltpu.VMEM((SUB,), jnp.int32),
            pltpu.VMEM((SUB,), jnp.int32),
            pltpu.VMEM((SUB,), jnp.int32),
            pltpu.VMEM((SUB,), jnp.int32),
            pltpu.SemaphoreType.DMA,
            pltpu.SemaphoreType.DMA,
            pltpu.SemaphoreType.DMA,
            pltpu.SemaphoreType.DMA,
        ],
    )


def _ff_block(xb, w1, w3, w2):
    a = lax.dot_general(xb, w1, (((1,), (1,)), ((), ())),
                        preferred_element_type=jnp.float32)
    g = lax.dot_general(xb, w3, (((1,), (1,)), ((), ())),
                        preferred_element_type=jnp.float32)
    h = (a * jax.nn.sigmoid(a)) * g
    return lax.dot_general(h, w2,
                           (((1,), (1,)), ((), ())),
                           preferred_element_type=jnp.float32)


def _w_copies(w1_hbm, w3_hbm, w2_hbm, w1b, w3b, w2b, sem, e, s):
    return (pltpu.make_async_copy(w1_hbm.at[e], w1b.at[s], sem),
            pltpu.make_async_copy(w3_hbm.at[e], w3b.at[s], sem),
            pltpu.make_async_copy(w2_hbm.at[e], w2b.at[s], sem))


def _gemm_body(bk_ref, xs_ref, w1_hbm, w3_hbm, w2_hbm, o_ref,
               w1b, w3b, w2b, sem):
    i = pl.program_id(0)
    fe = bk_ref[8 * i + 1]

    @pl.when(fe >= 0)
    def _():
        fs = bk_ref[8 * i + 2]
        pltpu.make_async_copy(w1_hbm.at[fe], w1b.at[fs], sem).start()
        pltpu.make_async_copy(w3_hbm.at[fe], w3b.at[fs], sem).start()
        pltpu.make_async_copy(w2_hbm.at[fe], w2b.at[fs], sem).start()

    @pl.when(bk_ref[8 * i + 3] == 1)
    def _():
        for c in _w_copies(w1_hbm, w3_hbm, w2_hbm, w1b, w3b, w2b, sem, 0, 0):
            c.wait()

    @pl.when(i < bk_ref[8 * i + 4])
    def _():
        s = bk_ref[8 * i]
        o_ref[...] = _ff_block(xs_ref[...], w1b[s], w3b[s], w2b[s])


def _gemm(bk, xs, w1b, w3b, w2b):
    grid_spec = pltpu.PrefetchScalarGridSpec(
        num_scalar_prefetch=1,
        grid=(NBLK,),
        in_specs=[
            pl.BlockSpec((BLK, DIM), lambda i, bk: (i, 0)),
            pl.BlockSpec(memory_space=pl.ANY),
            pl.BlockSpec(memory_space=pl.ANY),
            pl.BlockSpec(memory_space=pl.ANY),
        ],
        out_specs=pl.BlockSpec((BLK, DIM), lambda i, bk: (i, 0)),
        scratch_shapes=[
            pltpu.VMEM((2, HID, DIM), jnp.float32),
            pltpu.VMEM((2, HID, DIM), jnp.float32),
            pltpu.VMEM((2, DIM, HID), jnp.float32),
            pltpu.SemaphoreType.DMA,
        ],
    )
    return pl.pallas_call(
        _gemm_body,
        grid_spec=grid_spec,
        out_shape=jax.ShapeDtypeStruct((NPAD, DIM), jnp.float32),
        compiler_params=pltpu.CompilerParams(
            vmem_limit_bytes=100 * 1024 * 1024),
    )(bk, xs, w1b, w3b, w2b)


def _shared_body(x_ref, w1_ref, w3_ref, w2_ref, o_ref):
    o_ref[...] = _ff_block(x_ref[...], w1_ref[...], w3_ref[...], w2_ref[...])


def _shared(xf, sw1b, sw3b, sw2b):
    return pl.pallas_call(
        _shared_body,
        grid=(SEQ // SB,),
        in_specs=[
            pl.BlockSpec((SB, DIM), lambda i: (i, 0)),
            pl.BlockSpec((HID, DIM), lambda i: (0, 0)),
            pl.BlockSpec((HID, DIM), lambda i: (0, 0)),
            pl.BlockSpec((DIM, HID), lambda i: (0, 0)),
        ],
        out_specs=pl.BlockSpec((SB, DIM), lambda i: (i, 0)),
        out_shape=jax.ShapeDtypeStruct((SEQ, DIM), jnp.float32),
        compiler_params=pltpu.CompilerParams(
            vmem_limit_bytes=100 * 1024 * 1024),
    )(xf, sw1b, sw3b, sw2b)


SUBC = 8


def _combine_body(ys_hbm, sh_hbm, p0_hbm, p1_hbm, w0_hbm, w1_hbm, out_hbm,
                  acc_a, acc_b, r0_a, r0_b, r1_a, r1_b,
                  w0_a, w0_b, w1_a, w1_b, i0_a, i0_b, i1_a, i1_b,
                  lsem_a, lsem_b, osem_a, osem_b):
    wid = lax.axis_index("s") * NC + lax.axis_index("c")
    base = wid * TPW
    nch = TPW // SUBC
    accs = (acc_a, acc_b)
    r0s = (r0_a, r0_b)
    r1s = (r1_a, r1_b)
    w0s = (w0_a, w0_b)
    w1s = (w1_a, w1_b)
    i0s = (i0_a, i0_b)
    i1s = (i1_a, i1_b)
    lsems = (lsem_a, lsem_b)
    osems = (osem_a, osem_b)

    def load(g):
        b = base + g * SUBC
        s = g % 2
        if g % 2 == 0:
            p = (g // 2) % 2
            pltpu.sync_copy(w0_hbm.at[pl.ds(b, 2 * SUBC)], w0s[p])
            pltpu.sync_copy(w1_hbm.at[pl.ds(b, 2 * SUBC)], w1s[p])
        pltpu.sync_copy(p0_hbm.at[pl.ds(b, SUBC)], i0s[s])
        pltpu.sync_copy(p1_hbm.at[pl.ds(b, SUBC)], i1s[s])
        return (pltpu.async_copy(sh_hbm.at[pl.ds(b, SUBC)], accs[s], lsems[s]),
                pltpu.async_copy(ys_hbm.at[i0s[s]], r0s[s], lsems[s]),
                pltpu.async_copy(ys_hbm.at[i1s[s]], r1s[s], lsems[s]))

    loads = {0: load(0)}
    outs = {}
    for g in range(nch):
        s = g % 2
        if g + 1 < nch:
            if g >= 1:
                outs[g - 1].wait()
            loads[g + 1] = load(g + 1)
        for c in loads[g]:
            c.wait()
        w0vec = w0s[(g // 2) % 2][...]
        w1vec = w1s[(g // 2) % 2][...]
        off = (g % 2) * SUBC
        ws = [(w0vec[off + t], w1vec[off + t]) for t in range(SUBC)]
        acc, r0, r1 = accs[s], r0s[s], r1s[s]

        def col(c, carry, ws=ws, acc=acc, r0=r0, r1=r1):
            sl = pl.ds(c * 16, 16)
            for t in range(SUBC):
                a, gg = ws[t]
                acc[t, sl] = acc[t, sl] + a * r0[t, sl] + gg * r1[t, sl]
            return carry

        lax.fori_loop(0, DIM // 16, col, 0)
        outs[g] = pltpu.async_copy(acc, out_hbm.at[pl.ds(base + g * SUBC, SUBC)],
                                   osems[s])
    outs[nch - 2].wait()
    outs[nch - 1].wait()


@functools.cache
def _combine():
    return pl.kernel(
        _combine_body,
        out_type=jax.ShapeDtypeStruct((SEQ, DIM), jnp.float32),
        mesh=plsc.VectorSubcoreMesh(core_axis_name="c", subcore_axis_name="s",
                                    num_cores=NC, num_subcores=NS),
        scratch_types=[
            pltpu.VMEM((SUBC, DIM), jnp.float32),
            pltpu.VMEM((SUBC, DIM), jnp.float32),
            pltpu.VMEM((SUBC, DIM), jnp.float32),
            pltpu.VMEM((SUBC, DIM), jnp.float32),
            pltpu.VMEM((SUBC, DIM), jnp.float32),
            pltpu.VMEM((SUBC, DIM), jnp.float32),
            pltpu.VMEM((16,), jnp.float32),
            pltpu.VMEM((16,), jnp.float32),
            pltpu.VMEM((16,), jnp.float32),
            pltpu.VMEM((16,), jnp.float32),
            pltpu.VMEM((SUBC,), jnp.int32),
            pltpu.VMEM((SUBC,), jnp.int32),
            pltpu.VMEM((SUBC,), jnp.int32),
            pltpu.VMEM((SUBC,), jnp.int32),
            pltpu.SemaphoreType.DMA,
            pltpu.SemaphoreType.DMA,
            pltpu.SemaphoreType.DMA,
            pltpu.SemaphoreType.DMA,
        ],
    )


@jax.jit
def kernel(x, router_w, W1, W3, W2, sw1, sw3, sw2, experts_bias):
    xf = x.reshape(SEQ, DIM)
    pos0, pos1, w0, w1, bk2 = _route(xf, router_w,
                                     experts_bias.reshape(1, NE))
    bk = bk2.reshape(NBLK * 8)
    p0 = pos0.reshape(SEQ)
    p1 = pos1.reshape(SEQ)
    xs = _dispatch()(xf, p0, p1)
    ys = _gemm(bk, xs, W1, W3, W2)
    sh = _shared(xf, sw1, sw3, sw2)
    out = _combine()(ys, sh, p0, p1, w0.reshape(SEQ), w1.reshape(SEQ))
    return out.reshape(x.shape)
